# baseline (speedup 1.0000x reference)
import jax
import jax.numpy as jnp
from jax import lax
from jax.experimental import pallas as pl
from jax.experimental.pallas import tpu as pltpu
from jax.experimental.pallas import tpu_sc as plsc

NU = 25000
NI = 25000
NN = NU + NI
Q = 16
NC = 2
NT = 16
G = 1152
ADJ_G = 44
MM_G = 14
E_ADJ = ADJ_G * NT * G
E_MM = MM_G * NT * G
CH = 200
THIRD = 1.0 / 3.0


def _edge_pass(s, src_h, dst_h, val_h, tab_h, tab_mul, tab_off, ngrp, acc,
               sv2, vv2, dv2, rows2, sem_g, sem_s):
    def load_idx(g, b):
        ga = s * ngrp + g
        pltpu.sync_copy(src_h.at[pl.ds(ga * G, G)], sv2.at[b])
        pltpu.sync_copy(val_h.at[pl.ds(ga * G, G)], vv2.at[b])
        pltpu.sync_copy(dst_h.at[pl.ds(ga * (G // 128), G // 128)],
                        dv2.at[b])

        def addoff(i, c2):
            sl = pl.ds(i * 16, 16)
            sv2[b, sl] = sv2[b, sl] * tab_mul + tab_off
            return c2
        lax.fori_loop(0, G // 16, addoff, 0)

    def fire_gather(b):
        pltpu.async_copy(tab_h.at[sv2.at[b]], rows2.at[b], sem_g)

    def drain(sem, b):
        pltpu.make_async_copy(tab_h.at[pl.ds(0, G)], rows2.at[b], sem).wait()

    load_idx(0, 0)
    fire_gather(0)
    load_idx(1, 1)
    fire_gather(1)

    def grp(g, carry):
        b = lax.rem(g, 3)
        n2 = lax.rem(g + 2, 3)

        @pl.when(g > 0)
        def _():
            drain(sem_s, n2)

        @pl.when(g + 2 < ngrp)
        def _():
            load_idx(g + 2, n2)
            fire_gather(n2)

        drain(sem_g, b)

        def scale(gg, c2):
            vals16 = vv2[b, pl.ds(gg * 16, 16)]
            for e in range(16):
                bc = lax.gather(
                    vals16, jnp.full((16, 1), e, jnp.int32),
                    dimension_numbers=lax.GatherDimensionNumbers(
                        offset_dims=(), collapsed_slice_dims=(0,),
                        start_index_map=(0,)),
                    slice_sizes=(1,),
                    mode=lax.GatherScatterMode.PROMISE_IN_BOUNDS)
                r = gg * 16 + e
                rows2[b, r, pl.ds(0, 16)] = rows2[b, r, pl.ds(0, 16)] * bc
            return c2
        lax.fori_loop(0, G // 16, scale, 0)

        for j in range(G // 128):
            pltpu.async_copy(rows2.at[b, pl.ds(j * 128, 128)],
                             acc.at[dv2.at[b, j]], sem_s, add=True)
        return carry

    lax.fori_loop(0, ngrp, grp, 0)
    drain(sem_s, (ngrp - 1) % 3)


def _sc_body(ego_r, item_r, adj_src, adj_dst, adj_val,
             mm_src, mm_dst, mm_val,
             u4, i4, h_buf, e1_buf, e2_buf,
             sv2, vv2, dv2, rows2, b0, b1, b2, iv, iv2, acc,
             sem_g, sem_s, sem_h):
    c = lax.axis_index("c")
    s = lax.axis_index("s")
    n_chunks_i = NI // CH
    n_chunks_n = NN // CH

    z16 = jnp.zeros((16,), jnp.float32)

    def zrow(i, carry):
        b0[i, pl.ds(0, 16)] = z16
        return carry
    lax.fori_loop(0, CH, zrow, 0)

    def _chunked(n_chunks, fn):
        def fire(k, carry):
            ch = s + NT * k

            @pl.when(ch < n_chunks)
            def _():
                fn(ch).start()
            return carry
        lax.fori_loop(0, pl.cdiv(n_chunks, NT), fire, 0)

        def drain(k, carry):
            ch = s + NT * k

            @pl.when(ch < n_chunks)
            def _():
                fn(ch).wait()
            return carry
        lax.fori_loop(0, pl.cdiv(n_chunks, NT), drain, 0)

    def zero_acc(n_chunks):
        _chunked(n_chunks, lambda ch: pltpu.make_async_copy(
            b0, acc.at[pl.ds(ch * CH, CH)], sem_h))

    def write_out(n_chunks, out, out_base):
        _chunked(n_chunks, lambda ch: pltpu.make_async_copy(
            acc.at[pl.ds(ch * CH, CH)],
            out.at[pl.ds(out_base + ch * CH, CH)], sem_h))

    def quarter(p, carry):
        q = c * 2 + p
        zero_acc(n_chunks_n)
        plsc.subcore_barrier()

        _edge_pass(s, mm_src, mm_dst, mm_val, item_r, 4, q, MM_G, acc,
                   sv2, vv2, dv2, rows2, sem_g, sem_s)
        plsc.subcore_barrier()
        write_out(n_chunks_i, h_buf, q * NI)
        plsc.subcore_barrier()
        zero_acc(n_chunks_i)
        plsc.subcore_barrier()

        _edge_pass(s, adj_src, adj_dst, adj_val, ego_r, 4, q, ADJ_G,
                   acc, sv2, vv2, dv2, rows2, sem_g, sem_s)
        plsc.subcore_barrier()
        write_out(n_chunks_n, e1_buf, q * NN)
        plsc.subcore_barrier()
        zero_acc(n_chunks_n)
        plsc.subcore_barrier()

        _edge_pass(s, adj_src, adj_dst, adj_val, e1_buf, 1, q * NN, ADJ_G,
                   acc, sv2, vv2, dv2, rows2, sem_g, sem_s)
        plsc.subcore_barrier()
        write_out(n_chunks_n, e2_buf, q * NN)
        plsc.subcore_barrier()
        return carry

    lax.fori_loop(0, 2, quarter, 0)

    ar4 = jnp.arange(16, dtype=jnp.int32) * 4

    def scatter_out(out, base, cq):
        def bidx2(i, c3):
            iv2[0, pl.ds(i * 16, 16)] = (base + i * 16) * 4 + cq + ar4
            iv2[1, pl.ds(i * 16, 16)] = (base + 72 + i * 16) * 4 + cq + ar4
            return c3
        lax.fori_loop(0, 8, bidx2, 0)
        pltpu.sync_copy(b1.at[pl.ds(0, 128)], out.at[iv2.at[0]])
        pltpu.sync_copy(b1.at[pl.ds(72, 128)], out.at[iv2.at[1]])

    def combine(p, carry):
        q = c * 2 + p

        def cchunk(k, carry2):
            ch = s + NT * k

            @pl.when(ch < n_chunks_n)
            def _():
                r0 = ch * CH

                def bidx(i, c3):
                    iv[pl.ds(i * 16, 16)] = (r0 + i * 16) * 4 + q + ar4
                    return c3
                lax.fori_loop(0, (CH + 15) // 16, bidx, 0)
                d0 = pltpu.async_copy(ego_r.at[iv.at[pl.ds(0, CH)]], b0,
                                      sem_h)
                d1 = pltpu.make_async_copy(
                    e1_buf.at[pl.ds(q * NN + r0, CH)], b1, sem_h)
                d2 = pltpu.make_async_copy(
                    e2_buf.at[pl.ds(q * NN + r0, CH)], b2, sem_h)
                d1.start(), d2.start()
                pltpu.make_async_copy(ego_r.at[pl.ds(0, CH)], b0,
                                      sem_h).wait()
                d1.wait(), d2.wait()

                def mean_row(i, carry3):
                    b1[i, pl.ds(0, 16)] = (
                        b0[i, pl.ds(0, 16)] + b1[i, pl.ds(0, 16)]
                        + b2[i, pl.ds(0, 16)]) * THIRD
                    return carry3
                lax.fori_loop(0, CH, mean_row, 0)

                @pl.when(ch < n_chunks_i)
                def _():
                    scatter_out(u4, r0, q)

                @pl.when(ch >= n_chunks_i)
                def _():
                    ri = r0 - NU
                    pltpu.sync_copy(h_buf.at[pl.ds(q * NI + ri, CH)], b2)

                    def addh(i, carry3):
                        b1[i, pl.ds(0, 16)] = (b1[i, pl.ds(0, 16)]
                                               + b2[i, pl.ds(0, 16)])
                        return carry3
                    lax.fori_loop(0, CH, addh, 0)
                    scatter_out(i4, ri, q)
            return carry2
        lax.fori_loop(0, pl.cdiv(n_chunks_n, NT), cchunk, 0)
        return carry
    lax.fori_loop(0, 2, combine, 0)


@jax.jit
def kernel(adj_indices, adj_values, mm_indices, mm_values, user_emb, item_emb):
    f32 = jnp.float32
    i32 = jnp.int32

    ego_r = jnp.concatenate([user_emb, item_emb], axis=0).reshape(4 * NN, Q)
    item_r = item_emb.reshape(4 * NI, Q)

    def pad_edges(idx, val, e_pad, n_rows):
        pz = e_pad - idx.shape[1]
        spread = jnp.arange(pz, dtype=i32) % n_rows
        dst = jnp.concatenate([idx[0], spread])
        src = jnp.concatenate([idx[1], spread])
        v = jnp.concatenate([val, jnp.zeros((pz,), f32)])
        return src, dst.reshape(-1, 128), v

    adj_src, adj_dst, adj_val = pad_edges(adj_indices, adj_values, E_ADJ, NN)
    mm_src, mm_dst, mm_val = pad_edges(mm_indices, mm_values, E_MM, NI)

    mesh = plsc.VectorSubcoreMesh(core_axis_name="c", subcore_axis_name="s",
                                  num_cores=NC, num_subcores=NT)
    ker = pl.kernel(
        _sc_body,
        out_type=(
            jax.ShapeDtypeStruct((4 * NU, Q), f32),
            jax.ShapeDtypeStruct((4 * NI, Q), f32),
            jax.ShapeDtypeStruct((4 * NI, Q), f32),
            jax.ShapeDtypeStruct((4 * NN, Q), f32),
            jax.ShapeDtypeStruct((4 * NN, Q), f32),
        ),
        mesh=mesh,
        compiler_params=pltpu.CompilerParams(use_tc_tiling_on_sc=False),
        scratch_types=(
            pltpu.VMEM((3, G), i32),
            pltpu.VMEM((3, G), f32),
            pltpu.VMEM((3, G // 128, 128), i32),
            pltpu.VMEM((3, G, Q), f32),
            pltpu.VMEM((CH, Q), f32),
            pltpu.VMEM((CH, Q), f32),
            pltpu.VMEM((CH, Q), f32),
            pltpu.VMEM((208,), i32),
            pltpu.VMEM((2, 128), i32),
            pltpu.VMEM_SHARED((NN, Q), f32),
            pltpu.SemaphoreType.DMA,
            pltpu.SemaphoreType.DMA,
            pltpu.SemaphoreType.DMA,
        ),
    )
    u4, i4, _, _, _ = ker(ego_r, item_r, adj_src, adj_dst, adj_val,
                          mm_src, mm_dst, mm_val)
    return u4.reshape(NU, 4 * Q), i4.reshape(NI, 4 * Q)


# speedup vs baseline: 2.3869x; 2.3869x over previous
import jax
import jax.numpy as jnp
from jax import lax
from jax.experimental import pallas as pl
from jax.experimental.pallas import tpu as pltpu
from jax.experimental.pallas import tpu_sc as plsc

NU = 25000
NI = 25000
NN = NU + NI
Q = 16
NC = 2
NT = 16
G = 1792
ADJ_G = 28
MM_G = 9
E_ADJ = ADJ_G * NT * G
E_MM = MM_G * NT * G
CH = 200
THIRD = 1.0 / 3.0


def _edge_pass(s, src_h, dst_h, val_h, tab_h, tab_mul, tab_off, ngrp, acc,
               sv2, vv2, dv2, rows2, sem_g, sem_s):
    def load_sv(g, b):
        ga = s * ngrp + g
        pltpu.sync_copy(src_h.at[pl.ds(ga * G, G)], sv2.at[b])
        pltpu.sync_copy(val_h.at[pl.ds(ga * G, G)], vv2.at[b])

        def addoff(i, c2):
            sl = pl.ds(i * 16, 16)
            sv2[b, sl] = sv2[b, sl] * tab_mul + tab_off
            return c2
        lax.fori_loop(0, G // 16, addoff, 0)

    def load_dv(g, b):
        ga = s * ngrp + g
        pltpu.sync_copy(dst_h.at[pl.ds(ga * (G // 128), G // 128)],
                        dv2.at[b])

    def fire_gather(b):
        pltpu.async_copy(tab_h.at[sv2.at[b]], rows2.at[b], sem_g)

    def drain(sem, b):
        pltpu.make_async_copy(tab_h.at[pl.ds(0, G)], rows2.at[b], sem).wait()

    load_sv(0, 0)
    load_dv(0, 0)
    fire_gather(0)

    def grp(g, carry):
        b = g & 1

        @pl.when(g + 1 < ngrp)
        def _():
            load_sv(g + 1, 1 - b)

        @pl.when(g > 0)
        def _():
            drain(sem_s, 1 - b)

        @pl.when(g + 1 < ngrp)
        def _():
            load_dv(g + 1, 1 - b)

        drain(sem_g, b)

        @pl.when(g + 1 < ngrp)
        def _():
            fire_gather(1 - b)

        def scale(gg, c2):
            vals16 = vv2[b, pl.ds(gg * 16, 16)]
            for e in range(16):
                bc = lax.gather(
                    vals16, jnp.full((16, 1), e, jnp.int32),
                    dimension_numbers=lax.GatherDimensionNumbers(
                        offset_dims=(), collapsed_slice_dims=(0,),
                        start_index_map=(0,)),
                    slice_sizes=(1,),
                    mode=lax.GatherScatterMode.PROMISE_IN_BOUNDS)
                r = gg * 16 + e
                rows2[b, r, pl.ds(0, 16)] = rows2[b, r, pl.ds(0, 16)] * bc
            return c2
        lax.fori_loop(0, G // 16, scale, 0)

        for j in range(G // 128):
            pltpu.async_copy(rows2.at[b, pl.ds(j * 128, 128)],
                             acc.at[dv2.at[b, j]], sem_s, add=True)
        return carry

    lax.fori_loop(0, ngrp, grp, 0)
    drain(sem_s, (ngrp - 1) & 1)


def _sc_body(ego_r, item_r, adj_src, adj_dst, adj_val,
             mm_src, mm_dst, mm_val,
             u4, i4, h_buf, e1_buf, e2_buf,
             sv2, vv2, dv2, rows2, b0, b1, b2, iv, iv2, acc,
             sem_g, sem_s, sem_h):
    c = lax.axis_index("c")
    s = lax.axis_index("s")
    n_chunks_i = NI // CH
    n_chunks_n = NN // CH

    z16 = jnp.zeros((16,), jnp.float32)

    def zrow(i, carry):
        b0[i, pl.ds(0, 16)] = z16
        return carry
    lax.fori_loop(0, CH, zrow, 0)

    def _chunked(n_chunks, fn):
        def fire(k, carry):
            ch = s + NT * k

            @pl.when(ch < n_chunks)
            def _():
                fn(ch).start()
            return carry
        lax.fori_loop(0, pl.cdiv(n_chunks, NT), fire, 0)

        def drain(k, carry):
            ch = s + NT * k

            @pl.when(ch < n_chunks)
            def _():
                fn(ch).wait()
            return carry
        lax.fori_loop(0, pl.cdiv(n_chunks, NT), drain, 0)

    def zero_acc(n_chunks):
        _chunked(n_chunks, lambda ch: pltpu.make_async_copy(
            b0, acc.at[pl.ds(ch * CH, CH)], sem_h))

    def write_out(n_chunks, out, out_base):
        _chunked(n_chunks, lambda ch: pltpu.make_async_copy(
            acc.at[pl.ds(ch * CH, CH)],
            out.at[pl.ds(out_base + ch * CH, CH)], sem_h))

    def quarter(p, carry):
        q = c * 2 + p
        zero_acc(n_chunks_n)
        plsc.subcore_barrier()

        _edge_pass(s, mm_src, mm_dst, mm_val, item_r, 4, q, MM_G, acc,
                   sv2, vv2, dv2, rows2, sem_g, sem_s)
        plsc.subcore_barrier()
        write_out(n_chunks_i, h_buf, q * NI)
        plsc.subcore_barrier()
        zero_acc(n_chunks_i)
        plsc.subcore_barrier()

        _edge_pass(s, adj_src, adj_dst, adj_val, ego_r, 4, q, ADJ_G,
                   acc, sv2, vv2, dv2, rows2, sem_g, sem_s)
        plsc.subcore_barrier()
        write_out(n_chunks_n, e1_buf, q * NN)
        plsc.subcore_barrier()
        zero_acc(n_chunks_n)
        plsc.subcore_barrier()

        _edge_pass(s, adj_src, adj_dst, adj_val, e1_buf, 1, q * NN, ADJ_G,
                   acc, sv2, vv2, dv2, rows2, sem_g, sem_s)
        plsc.subcore_barrier()
        write_out(n_chunks_n, e2_buf, q * NN)
        plsc.subcore_barrier()
        return carry

    lax.fori_loop(0, 2, quarter, 0)

    ar4 = jnp.arange(16, dtype=jnp.int32) * 4

    def scatter_out(out, base, cq):
        def bidx2(i, c3):
            iv2[0, pl.ds(i * 16, 16)] = (base + i * 16) * 4 + cq + ar4
            iv2[1, pl.ds(i * 16, 16)] = (base + 72 + i * 16) * 4 + cq + ar4
            return c3
        lax.fori_loop(0, 8, bidx2, 0)
        pltpu.sync_copy(b1.at[pl.ds(0, 128)], out.at[iv2.at[0]])
        pltpu.sync_copy(b1.at[pl.ds(72, 128)], out.at[iv2.at[1]])

    def combine(p, carry):
        q = c * 2 + p

        def cchunk(k, carry2):
            ch = s + NT * k

            @pl.when(ch < n_chunks_n)
            def _():
                r0 = ch * CH

                def bidx(i, c3):
                    iv[pl.ds(i * 16, 16)] = (r0 + i * 16) * 4 + q + ar4
                    return c3
                lax.fori_loop(0, (CH + 15) // 16, bidx, 0)
                d0 = pltpu.async_copy(ego_r.at[iv.at[pl.ds(0, CH)]], b0,
                                      sem_h)
                d1 = pltpu.make_async_copy(
                    e1_buf.at[pl.ds(q * NN + r0, CH)], b1, sem_h)
                d2 = pltpu.make_async_copy(
                    e2_buf.at[pl.ds(q * NN + r0, CH)], b2, sem_h)
                d1.start(), d2.start()
                pltpu.make_async_copy(ego_r.at[pl.ds(0, CH)], b0,
                                      sem_h).wait()
                d1.wait(), d2.wait()

                def mean_row(i, carry3):
                    b1[i, pl.ds(0, 16)] = (
                        b0[i, pl.ds(0, 16)] + b1[i, pl.ds(0, 16)]
                        + b2[i, pl.ds(0, 16)]) * THIRD
                    return carry3
                lax.fori_loop(0, CH, mean_row, 0)

                @pl.when(ch < n_chunks_i)
                def _():
                    scatter_out(u4, r0, q)

                @pl.when(ch >= n_chunks_i)
                def _():
                    ri = r0 - NU
                    pltpu.sync_copy(h_buf.at[pl.ds(q * NI + ri, CH)], b2)

                    def addh(i, carry3):
                        b1[i, pl.ds(0, 16)] = (b1[i, pl.ds(0, 16)]
                                               + b2[i, pl.ds(0, 16)])
                        return carry3
                    lax.fori_loop(0, CH, addh, 0)
                    scatter_out(i4, ri, q)
            return carry2
        lax.fori_loop(0, pl.cdiv(n_chunks_n, NT), cchunk, 0)
        return carry
    lax.fori_loop(0, 2, combine, 0)


@jax.jit
def kernel(adj_indices, adj_values, mm_indices, mm_values, user_emb, item_emb):
    f32 = jnp.float32
    i32 = jnp.int32

    ego_r = jnp.concatenate([user_emb, item_emb], axis=0).reshape(4 * NN, Q)
    item_r = item_emb.reshape(4 * NI, Q)

    def pad_edges(idx, val, e_pad, n_rows):
        pz = e_pad - idx.shape[1]
        spread = jnp.arange(pz, dtype=i32) % n_rows
        dst = jnp.concatenate([idx[0], spread])
        src = jnp.concatenate([idx[1], spread])
        v = jnp.concatenate([val, jnp.zeros((pz,), f32)])
        return src, dst.reshape(-1, 128), v

    adj_src, adj_dst, adj_val = pad_edges(adj_indices, adj_values, E_ADJ, NN)
    mm_src, mm_dst, mm_val = pad_edges(mm_indices, mm_values, E_MM, NI)

    mesh = plsc.VectorSubcoreMesh(core_axis_name="c", subcore_axis_name="s",
                                  num_cores=NC, num_subcores=NT)
    ker = pl.kernel(
        _sc_body,
        out_type=(
            jax.ShapeDtypeStruct((4 * NU, Q), f32),
            jax.ShapeDtypeStruct((4 * NI, Q), f32),
            jax.ShapeDtypeStruct((4 * NI, Q), f32),
            jax.ShapeDtypeStruct((4 * NN, Q), f32),
            jax.ShapeDtypeStruct((4 * NN, Q), f32),
        ),
        mesh=mesh,
        compiler_params=pltpu.CompilerParams(use_tc_tiling_on_sc=False),
        scratch_types=(
            pltpu.VMEM((2, G), i32),
            pltpu.VMEM((2, G), f32),
            pltpu.VMEM((2, G // 128, 128), i32),
            pltpu.VMEM((2, G, Q), f32),
            pltpu.VMEM((CH, Q), f32),
            pltpu.VMEM((CH, Q), f32),
            pltpu.VMEM((CH, Q), f32),
            pltpu.VMEM((208,), i32),
            pltpu.VMEM((2, 128), i32),
            pltpu.VMEM_SHARED((NN, Q), f32),
            pltpu.SemaphoreType.DMA,
            pltpu.SemaphoreType.DMA,
            pltpu.SemaphoreType.DMA,
        ),
    )
    u4, i4, _, _, _ = ker(ego_r, item_r, adj_src, adj_dst, adj_val,
                          mm_src, mm_dst, mm_val)
    return u4.reshape(NU, 4 * Q), i4.reshape(NI, 4 * Q)


# speedup vs baseline: 2.4274x; 1.0170x over previous
import jax
import jax.numpy as jnp
from jax import lax
from jax.experimental import pallas as pl
from jax.experimental.pallas import tpu as pltpu
from jax.experimental.pallas import tpu_sc as plsc

NU = 25000
NI = 25000
NN = NU + NI
Q = 16
NC = 2
NT = 16
G = 1792
ADJ_G = 28
MM_G = 9
E_ADJ = ADJ_G * NT * G
E_MM = MM_G * NT * G
CH = 200
THIRD = 1.0 / 3.0


def _edge_pass(s, src_h, dst_h, val_h, tab_h, tab_mul, tab_off, ngrp, acc,
               sv2, vv2, dv2, rows2, sem_g, sem_s):
    def load_sv(g, b):
        ga = s * ngrp + g
        pltpu.sync_copy(src_h.at[pl.ds(ga * G, G)], sv2.at[b])
        pltpu.sync_copy(val_h.at[pl.ds(ga * G, G)], vv2.at[b])

        def addoff(i, c2):
            sl = pl.ds(i * 16, 16)
            sv2[b, sl] = sv2[b, sl] * tab_mul + tab_off
            return c2
        lax.fori_loop(0, G // 16, addoff, 0)

    def load_dv(g, b):
        ga = s * ngrp + g
        pltpu.sync_copy(dst_h.at[pl.ds(ga * (G // 128), G // 128)],
                        dv2.at[b])

    def fire_gather(b):
        pltpu.async_copy(tab_h.at[sv2.at[b]], rows2.at[b], sem_g)

    def drain(sem, b):
        pltpu.make_async_copy(tab_h.at[pl.ds(0, G)], rows2.at[b], sem).wait()

    load_sv(0, 0)
    load_dv(0, 0)
    fire_gather(0)

    def grp(g, carry):
        b = g & 1

        @pl.when(g + 1 < ngrp)
        def _():
            load_sv(g + 1, 1 - b)

        @pl.when(g > 0)
        def _():
            drain(sem_s, 1 - b)

        @pl.when(g + 1 < ngrp)
        def _():
            load_dv(g + 1, 1 - b)

        drain(sem_g, b)

        @pl.when(g + 1 < ngrp)
        def _():
            fire_gather(1 - b)

        def scale(gg, c2):
            vals16 = vv2[b, pl.ds(gg * 16, 16)]
            for e in range(16):
                bc = lax.gather(
                    vals16, jnp.full((16, 1), e, jnp.int32),
                    dimension_numbers=lax.GatherDimensionNumbers(
                        offset_dims=(), collapsed_slice_dims=(0,),
                        start_index_map=(0,)),
                    slice_sizes=(1,),
                    mode=lax.GatherScatterMode.PROMISE_IN_BOUNDS)
                r = gg * 16 + e
                rows2[b, r, pl.ds(0, 16)] = rows2[b, r, pl.ds(0, 16)] * bc
            return c2
        lax.fori_loop(0, G // 16, scale, 0)

        for j in range(G // 128):
            pltpu.async_copy(rows2.at[b, pl.ds(j * 128, 128)],
                             acc.at[dv2.at[b, j]], sem_s, add=True)
        return carry

    lax.fori_loop(0, ngrp, grp, 0)
    drain(sem_s, (ngrp - 1) & 1)


def _sc_body(ego_r, item_r, adj_src, adj_dst, adj_val,
             mm_src, mm_dst, mm_val,
             u4, i4, e1_buf, e2_buf,
             sv2, vv2, dv2, rows2, b0, b1, b2, iv, iv2, acc,
             sem_g, sem_s, sem_h):
    c = lax.axis_index("c")
    s = lax.axis_index("s")
    n_chunks_i = NI // CH
    n_chunks_n = NN // CH

    z16 = jnp.zeros((16,), jnp.float32)

    def zrow(i, carry):
        b0[i, pl.ds(0, 16)] = z16
        return carry
    lax.fori_loop(0, CH, zrow, 0)

    def _chunked(n_chunks, fn):
        def fire(k, carry):
            ch = s + NT * k

            @pl.when(ch < n_chunks)
            def _():
                fn(ch).start()
            return carry
        lax.fori_loop(0, pl.cdiv(n_chunks, NT), fire, 0)

        def drain(k, carry):
            ch = s + NT * k

            @pl.when(ch < n_chunks)
            def _():
                fn(ch).wait()
            return carry
        lax.fori_loop(0, pl.cdiv(n_chunks, NT), drain, 0)

    def zero_acc(n_chunks):
        _chunked(n_chunks, lambda ch: pltpu.make_async_copy(
            b0, acc.at[pl.ds(ch * CH, CH)], sem_h))

    def write_out(n_chunks, out, out_base):
        _chunked(n_chunks, lambda ch: pltpu.make_async_copy(
            acc.at[pl.ds(ch * CH, CH)],
            out.at[pl.ds(out_base + ch * CH, CH)], sem_h))

    ar4 = jnp.arange(16, dtype=jnp.int32) * 4

    def scatter_out(out, base, cq):
        def bidx2(i, c3):
            iv2[0, pl.ds(i * 16, 16)] = (base + i * 16) * 4 + cq + ar4
            iv2[1, pl.ds(i * 16, 16)] = (base + 72 + i * 16) * 4 + cq + ar4
            return c3
        lax.fori_loop(0, 8, bidx2, 0)
        da = pltpu.async_copy(b1.at[pl.ds(0, 128)], out.at[iv2.at[0]], sem_h)
        db = pltpu.async_copy(b1.at[pl.ds(72, 128)], out.at[iv2.at[1]], sem_h)
        da.wait(), db.wait()

    def quarter(p, carry):
        q = c * 2 + p
        zero_acc(n_chunks_n)
        plsc.subcore_barrier()

        _edge_pass(s, adj_src, adj_dst, adj_val, ego_r, 4, q, ADJ_G,
                   acc, sv2, vv2, dv2, rows2, sem_g, sem_s)
        plsc.subcore_barrier()
        write_out(n_chunks_n, e1_buf, q * NN)
        plsc.subcore_barrier()
        zero_acc(n_chunks_n)
        plsc.subcore_barrier()

        _edge_pass(s, adj_src, adj_dst, adj_val, e1_buf, 1, q * NN, ADJ_G,
                   acc, sv2, vv2, dv2, rows2, sem_g, sem_s)
        plsc.subcore_barrier()
        write_out(n_chunks_n, e2_buf, q * NN)
        plsc.subcore_barrier()
        zero_acc(n_chunks_i)
        plsc.subcore_barrier()

        _edge_pass(s, mm_src, mm_dst, mm_val, item_r, 4, q, MM_G, acc,
                   sv2, vv2, dv2, rows2, sem_g, sem_s)
        plsc.subcore_barrier()

        def cchunk(k, carry2):
            ch = s + NT * k

            @pl.when(ch < n_chunks_n)
            def _():
                r0 = ch * CH

                def bidx(i, c3):
                    iv[pl.ds(i * 16, 16)] = (r0 + i * 16) * 4 + q + ar4
                    return c3
                lax.fori_loop(0, (CH + 15) // 16, bidx, 0)
                d0 = pltpu.async_copy(ego_r.at[iv.at[pl.ds(0, CH)]], b0,
                                      sem_h)
                d1 = pltpu.make_async_copy(
                    e1_buf.at[pl.ds(q * NN + r0, CH)], b1, sem_h)
                d2 = pltpu.make_async_copy(
                    e2_buf.at[pl.ds(q * NN + r0, CH)], b2, sem_h)
                d1.start(), d2.start()
                pltpu.make_async_copy(ego_r.at[pl.ds(0, CH)], b0,
                                      sem_h).wait()
                d1.wait(), d2.wait()

                def mean_row(i, carry3):
                    b1[i, pl.ds(0, 16)] = (
                        b0[i, pl.ds(0, 16)] + b1[i, pl.ds(0, 16)]
                        + b2[i, pl.ds(0, 16)]) * THIRD
                    return carry3
                lax.fori_loop(0, CH, mean_row, 0)

                @pl.when(ch < n_chunks_i)
                def _():
                    scatter_out(u4, r0, q)

                @pl.when(ch >= n_chunks_i)
                def _():
                    ri = r0 - NU
                    pltpu.sync_copy(acc.at[pl.ds(ri, CH)], b2)

                    def addh(i, carry3):
                        b1[i, pl.ds(0, 16)] = (b1[i, pl.ds(0, 16)]
                                               + b2[i, pl.ds(0, 16)])
                        return carry3
                    lax.fori_loop(0, CH, addh, 0)
                    scatter_out(i4, ri, q)
            return carry2
        lax.fori_loop(0, pl.cdiv(n_chunks_n, NT), cchunk, 0)
        plsc.subcore_barrier()
        return carry

    lax.fori_loop(0, 2, quarter, 0)


@jax.jit
def kernel(adj_indices, adj_values, mm_indices, mm_values, user_emb, item_emb):
    f32 = jnp.float32
    i32 = jnp.int32

    ego_r = jnp.concatenate([user_emb, item_emb], axis=0).reshape(4 * NN, Q)
    item_r = item_emb.reshape(4 * NI, Q)

    def pad_edges(idx, val, e_pad, n_rows):
        pz = e_pad - idx.shape[1]
        spread = jnp.arange(pz, dtype=i32) % n_rows
        dst = jnp.concatenate([idx[0], spread])
        src = jnp.concatenate([idx[1], spread])
        v = jnp.concatenate([val, jnp.zeros((pz,), f32)])
        return src, dst.reshape(-1, 128), v

    adj_src, adj_dst, adj_val = pad_edges(adj_indices, adj_values, E_ADJ, NN)
    mm_src, mm_dst, mm_val = pad_edges(mm_indices, mm_values, E_MM, NI)

    mesh = plsc.VectorSubcoreMesh(core_axis_name="c", subcore_axis_name="s",
                                  num_cores=NC, num_subcores=NT)
    ker = pl.kernel(
        _sc_body,
        out_type=(
            jax.ShapeDtypeStruct((4 * NU, Q), f32),
            jax.ShapeDtypeStruct((4 * NI, Q), f32),
            jax.ShapeDtypeStruct((4 * NN, Q), f32),
            jax.ShapeDtypeStruct((4 * NN, Q), f32),
        ),
        mesh=mesh,
        compiler_params=pltpu.CompilerParams(use_tc_tiling_on_sc=False),
        scratch_types=(
            pltpu.VMEM((2, G), i32),
            pltpu.VMEM((2, G), f32),
            pltpu.VMEM((2, G // 128, 128), i32),
            pltpu.VMEM((2, G, Q), f32),
            pltpu.VMEM((CH, Q), f32),
            pltpu.VMEM((CH, Q), f32),
            pltpu.VMEM((CH, Q), f32),
            pltpu.VMEM((208,), i32),
            pltpu.VMEM((2, 128), i32),
            pltpu.VMEM_SHARED((NN, Q), f32),
            pltpu.SemaphoreType.DMA,
            pltpu.SemaphoreType.DMA,
            pltpu.SemaphoreType.DMA,
        ),
    )
    u4, i4, _, _ = ker(ego_r, item_r, adj_src, adj_dst, adj_val,
                       mm_src, mm_dst, mm_val)
    return u4.reshape(NU, 4 * Q), i4.reshape(NI, 4 * Q)
